# baseline (speedup 1.0000x reference)
import functools

import jax
import jax.numpy as jnp
from jax import lax
from jax.experimental import pallas as pl
from jax.experimental.pallas import tpu as pltpu
from jax.experimental.pallas import tpu_sc as plsc

D_MODEL = 64
DP = 128


@functools.cache
def _make_gather(B: int, V: int):
    info = plsc.get_sparse_core_info()
    NC, NS = info.num_cores, info.num_subcores
    NW = NC * NS
    assert B % NW == 0
    b_per_w = B // NW
    C = 200
    NBUF = 4
    LAG = 2
    assert b_per_w % C == 0 and (C % 8 == 0 or (C * NW) % 8 == 0)
    n_chunks = b_per_w // C
    assert n_chunks % NBUF == 0 and n_chunks > NBUF

    mesh = plsc.VectorSubcoreMesh(core_axis_name="c", subcore_axis_name="s")

    @functools.partial(
        pl.kernel,
        mesh=mesh,
        out_type=jax.ShapeDtypeStruct((B, DP), jnp.float32),
        scratch_types=[
            pltpu.VMEM((b_per_w,), jnp.int32),
            pltpu.VMEM((NBUF, C, DP), jnp.float32),
            pltpu.SemaphoreType.DMA,
            pltpu.SemaphoreType.DMA((NBUF,)),
            pltpu.SemaphoreType.DMA((NBUF,)),
        ],
    )
    def gather_kernel(idx_hbm, table_hbm, out_hbm, idx_v, rows_v, sem_i, sem_g, sem_o):
        wid = lax.axis_index("s") * NC + lax.axis_index("c")
        base = wid * b_per_w

        pltpu.async_copy(idx_hbm.at[pl.ds(base, b_per_w)], idx_v, sem_i).wait()

        def start_gather(i, b):
            pltpu.async_copy(
                table_hbm.at[idx_v.at[pl.ds(i * C, C)]], rows_v.at[b], sem_g.at[b]
            )

        def finish_gather_start_writeback(i, b):
            pltpu.make_async_copy(
                table_hbm.at[idx_v.at[pl.ds(i * C, C)]], rows_v.at[b], sem_g.at[b]
            ).wait()
            pltpu.async_copy(
                rows_v.at[b], out_hbm.at[pl.ds(base + i * C, C)], sem_o.at[b]
            )

        @pl.loop(0, n_chunks, step=NBUF)
        def _(g):
            for b in range(NBUF):
                i = g + b

                @pl.when(i >= NBUF)
                def _():
                    pltpu.make_async_copy(
                        rows_v.at[b], out_hbm.at[pl.ds(base, C)], sem_o.at[b]
                    ).wait()

                start_gather(i, b)

                @pl.when(i >= LAG)
                def _():
                    finish_gather_start_writeback(i - LAG, (b - LAG) % NBUF)

        for j in range(LAG):
            i = n_chunks - LAG + j
            finish_gather_start_writeback(i, i % NBUF)
        for b in range(NBUF):
            pltpu.make_async_copy(
                rows_v.at[b], out_hbm.at[pl.ds(base, C)], sem_o.at[b]
            ).wait()

    return gather_kernel


def kernel(x, table):
    B = x.shape[0] * x.shape[1]
    table_p = jnp.pad(table, ((0, 0), (0, DP - D_MODEL)))
    out = _make_gather(B, table.shape[0])(x.reshape(B), table_p)
    return out.reshape(x.shape[0], x.shape[1], DP)[:, :, :D_MODEL]
